# Initial kernel scaffold; baseline (speedup 1.0000x reference)
import functools

import jax
import jax.numpy as jnp
from jax import lax
from jax.experimental import pallas as pl
from jax.experimental.pallas import tpu as pltpu
from jax.experimental.pallas import tpu_sc as plsc

N = 10000
E = 320000
D = 128
NC = 2
NS = 16
EPC = E // NC
EPW = EPC // NS
K = 128
NFULL = EPW // K
TAIL = EPW - NFULL * K
RPW = N // NS
DEGW = 16


def _edge_pass(h_hbm, src_hbm, dst_hbm, agg_sh, src_v, dst_v, rows_v, gsem,
               base, nchunks, ksz, deg_sh=None, ones_v=None):
    def body(i, _):
        off = base + i * ksz
        pltpu.sync_copy(src_hbm.at[pl.ds(off, ksz)], src_v)
        pltpu.sync_copy(dst_hbm.at[pl.ds(off, ksz)], dst_v)
        pltpu.async_copy(h_hbm.at[src_v], rows_v, gsem).wait()
        pltpu.sync_copy(rows_v, agg_sh.at[dst_v], add=True)
        if deg_sh is not None:
            pltpu.sync_copy(ones_v, deg_sh.at[dst_v], add=True)
        return 0

    lax.fori_loop(0, nchunks, body, 0)


def _sc_agg_deg_body(h_hbm, src_hbm, dst_hbm, zeros_hbm, degz_hbm, ones_hbm,
                     agg_out, deg_out,
                     src_v, dst_v, rows_v, src_t, dst_t, rows_t,
                     ones_v, ones_t, gsem, agg_sh, deg_sh):
    cid = lax.axis_index("c")
    sid = lax.axis_index("s")
    pltpu.sync_copy(zeros_hbm.at[pl.ds(sid * RPW, RPW)],
                    agg_sh.at[pl.ds(sid * RPW, RPW)])
    pltpu.sync_copy(degz_hbm.at[pl.ds(sid * RPW, RPW)],
                    deg_sh.at[pl.ds(sid * RPW, RPW)])
    pltpu.sync_copy(ones_hbm.at[pl.ds(0, K)], ones_v)
    pltpu.sync_copy(ones_hbm.at[pl.ds(0, TAIL)], ones_t)
    plsc.subcore_barrier()

    base = cid * EPC + sid * EPW
    _edge_pass(h_hbm, src_hbm, dst_hbm, agg_sh, src_v, dst_v, rows_v, gsem,
               base, NFULL, K, deg_sh=deg_sh, ones_v=ones_v)
    _edge_pass(h_hbm, src_hbm, dst_hbm, agg_sh, src_t, dst_t, rows_t, gsem,
               base + NFULL * K, 1, TAIL, deg_sh=deg_sh, ones_v=ones_t)
    plsc.subcore_barrier()

    row0 = cid * N + sid * RPW
    pltpu.sync_copy(agg_sh.at[pl.ds(sid * RPW, RPW)],
                    agg_out.at[pl.ds(row0, RPW)])
    pltpu.sync_copy(deg_sh.at[pl.ds(sid * RPW, RPW)],
                    deg_out.at[pl.ds(row0, RPW)])


def _sc_agg_body(h_hbm, src_hbm, dst_hbm, zeros_hbm,
                 agg_out,
                 src_v, dst_v, rows_v, src_t, dst_t, rows_t, gsem, agg_sh):
    cid = lax.axis_index("c")
    sid = lax.axis_index("s")
    pltpu.sync_copy(zeros_hbm.at[pl.ds(sid * RPW, RPW)],
                    agg_sh.at[pl.ds(sid * RPW, RPW)])
    plsc.subcore_barrier()

    base = cid * EPC + sid * EPW
    _edge_pass(h_hbm, src_hbm, dst_hbm, agg_sh, src_v, dst_v, rows_v, gsem,
               base, NFULL, K)
    _edge_pass(h_hbm, src_hbm, dst_hbm, agg_sh, src_t, dst_t, rows_t, gsem,
               base + NFULL * K, 1, TAIL)
    plsc.subcore_barrier()

    row0 = cid * N + sid * RPW
    pltpu.sync_copy(agg_sh.at[pl.ds(sid * RPW, RPW)],
                    agg_out.at[pl.ds(row0, RPW)])


_SC_MESH = plsc.VectorSubcoreMesh(core_axis_name="c", subcore_axis_name="s")

_agg_deg_call = pl.kernel(
    _sc_agg_deg_body,
    out_type=(jax.ShapeDtypeStruct((NC * N, D), jnp.float32),
              jax.ShapeDtypeStruct((NC * N, DEGW), jnp.float32)),
    mesh=_SC_MESH,
    scratch_types=[
        pltpu.VMEM((K,), jnp.int32),
        pltpu.VMEM((K,), jnp.int32),
        pltpu.VMEM((K, D), jnp.float32),
        pltpu.VMEM((TAIL,), jnp.int32),
        pltpu.VMEM((TAIL,), jnp.int32),
        pltpu.VMEM((TAIL, D), jnp.float32),
        pltpu.VMEM((K, DEGW), jnp.float32),
        pltpu.VMEM((TAIL, DEGW), jnp.float32),
        pltpu.SemaphoreType.DMA,
        pltpu.VMEM_SHARED((N, D), jnp.float32),
        pltpu.VMEM_SHARED((N, DEGW), jnp.float32),
    ],
)

_agg_call = pl.kernel(
    _sc_agg_body,
    out_type=jax.ShapeDtypeStruct((NC * N, D), jnp.float32),
    mesh=_SC_MESH,
    scratch_types=[
        pltpu.VMEM((K,), jnp.int32),
        pltpu.VMEM((K,), jnp.int32),
        pltpu.VMEM((K, D), jnp.float32),
        pltpu.VMEM((TAIL,), jnp.int32),
        pltpu.VMEM((TAIL,), jnp.int32),
        pltpu.VMEM((TAIL, D), jnp.float32),
        pltpu.SemaphoreType.DMA,
        pltpu.VMEM_SHARED((N, D), jnp.float32),
    ],
)

BN = 1000


def _tc_mlp_body(scale_ref, h_ref, a0_ref, a1_ref, d0_ref, d1_ref,
                 w1_ref, b1_ref, w2_ref, b2_ref, o_ref):
    deg = jnp.maximum(d0_ref[:, :1] + d1_ref[:, :1], 1.0)
    z = scale_ref[0, 0] * h_ref[...] + (a0_ref[...] + a1_ref[...]) / deg
    z = jnp.dot(z, w1_ref[...], preferred_element_type=jnp.float32) + b1_ref[...]
    z = jnp.where(z > 0, z, 0.01 * z)
    z = jnp.dot(z, w2_ref[...], preferred_element_type=jnp.float32) + b2_ref[...]
    o_ref[...] = jnp.where(z > 0, z, 0.01 * z)


_NB = N // BN

_tc_mlp_call = pl.pallas_call(
    _tc_mlp_body,
    grid=(_NB,),
    in_specs=[
        pl.BlockSpec(memory_space=pltpu.SMEM),
        pl.BlockSpec((BN, D), lambda i: (i, 0)),
        pl.BlockSpec((BN, D), lambda i: (i, 0)),
        pl.BlockSpec((BN, D), lambda i: (i + _NB, 0)),
        pl.BlockSpec((BN, DEGW), lambda i: (i, 0)),
        pl.BlockSpec((BN, DEGW), lambda i: (i + _NB, 0)),
        pl.BlockSpec((D, D), lambda i: (0, 0)),
        pl.BlockSpec((1, D), lambda i: (0, 0)),
        pl.BlockSpec((D, D), lambda i: (0, 0)),
        pl.BlockSpec((1, D), lambda i: (0, 0)),
    ],
    out_specs=pl.BlockSpec((BN, D), lambda i: (i, 0)),
    out_shape=jax.ShapeDtypeStruct((N, D), jnp.float32),
)


def kernel(x, edge_index,
           eps0, W1_0, b1_0, W2_0, b2_0,
           eps1, W1_1, b1_1, W2_1, b2_1,
           eps2, W1_2, b1_2, W2_2, b2_2):
    src = edge_index[0]
    dst = edge_index[1]
    zeros = jnp.zeros((N, D), jnp.float32)
    degz = jnp.zeros((N, DEGW), jnp.float32)
    ones = jnp.ones((K, DEGW), jnp.float32)

    def mlp(h, agg2, deg2, eps, W1, b1, W2, b2):
        scale = (1.0 + eps).reshape(1, 1)
        return _tc_mlp_call(scale, h, agg2, agg2, deg2, deg2,
                            W1, b1.reshape(1, D), W2, b2.reshape(1, D))

    agg2, deg2 = _agg_deg_call(x, src, dst, zeros, degz, ones)
    h = mlp(x, agg2, deg2, eps0, W1_0, b1_0, W2_0, b2_0)
    agg2 = _agg_call(h, src, dst, zeros)
    h = mlp(h, agg2, deg2, eps1, W1_1, b1_1, W2_1, b2_1)
    agg2 = _agg_call(h, src, dst, zeros)
    h = mlp(h, agg2, deg2, eps2, W1_2, b1_2, W2_2, b2_2)
    return h


# speedup vs baseline: 5.3325x; 5.3325x over previous
import jax
import jax.numpy as jnp
from jax import lax
from jax.experimental import pallas as pl
from jax.experimental.pallas import tpu as pltpu
from jax.experimental.pallas import tpu_sc as plsc

N = 10000
E = 320000
D = 128
NC = 2
NS = 16
EPC = E // NC
EPW = EPC // NS
K = 128
NFULL = EPW // K
TAIL = EPW - NFULL * K
RSTRIPE = 624


def _striped_rows(sid, copy_fn):
    r0 = sid * RSTRIPE
    for j, nr in enumerate((128, 128, 128, 128, 112)):
        copy_fn(r0 + j * 128, nr)

    @pl.when(sid == NS - 1)
    def _():
        copy_fn(NS * RSTRIPE, N - NS * RSTRIPE)


def _sc_agg_body(h_hbm, src_hbm, dst_hbm, zeros_hbm,
                 agg_out,
                 src_v, dst_v, rows_v, src_t, dst_t, rows_t, gsem, agg_sh):
    cid = lax.axis_index("c")
    sid = lax.axis_index("s")

    pltpu.sync_copy(zeros_hbm.at[pl.ds(0, K)], rows_v)
    _striped_rows(sid, lambda r0, nr: pltpu.sync_copy(
        rows_v.at[pl.ds(0, nr)], agg_sh.at[pl.ds(r0, nr)]))
    plsc.subcore_barrier()

    base = cid * EPC + sid * EPW

    def chunk(i, _):
        off = base + i * K
        pltpu.sync_copy(src_hbm.at[pl.ds(off, K)], src_v)
        pltpu.sync_copy(dst_hbm.at[pl.ds(off, K)], dst_v)
        pltpu.async_copy(h_hbm.at[src_v], rows_v, gsem).wait()
        pltpu.sync_copy(rows_v, agg_sh.at[dst_v], add=True)
        return 0

    lax.fori_loop(0, NFULL, chunk, 0)

    toff = base + NFULL * K
    pltpu.sync_copy(src_hbm.at[pl.ds(toff, TAIL)], src_t)
    pltpu.sync_copy(dst_hbm.at[pl.ds(toff, TAIL)], dst_t)
    pltpu.async_copy(h_hbm.at[src_t], rows_t, gsem).wait()
    pltpu.sync_copy(rows_t, agg_sh.at[dst_t], add=True)
    plsc.subcore_barrier()

    def out_stripe(r0, nr):
        pltpu.sync_copy(agg_sh.at[pl.ds(r0, nr)], rows_v.at[pl.ds(0, nr)])
        pltpu.sync_copy(rows_v.at[pl.ds(0, nr)],
                        agg_out.at[pl.ds(cid * N + r0, nr)])

    _striped_rows(sid, out_stripe)


def _sc_deg_body(dst_hbm, zeros_hbm, ones_hbm,
                 deg_out,
                 dst_v, dst_t, ones_v, rows_v, deg_sh):
    cid = lax.axis_index("c")
    sid = lax.axis_index("s")

    pltpu.sync_copy(zeros_hbm.at[pl.ds(0, K)], rows_v)
    pltpu.sync_copy(ones_hbm.at[pl.ds(0, K)], ones_v)
    _striped_rows(sid, lambda r0, nr: pltpu.sync_copy(
        rows_v.at[pl.ds(0, nr)], deg_sh.at[pl.ds(r0, nr)]))
    plsc.subcore_barrier()

    base = cid * EPC + sid * EPW

    def chunk(i, _):
        pltpu.sync_copy(dst_hbm.at[pl.ds(base + i * K, K)], dst_v)
        pltpu.sync_copy(ones_v, deg_sh.at[dst_v], add=True)
        return 0

    lax.fori_loop(0, NFULL, chunk, 0)

    pltpu.sync_copy(dst_hbm.at[pl.ds(base + NFULL * K, TAIL)], dst_t)
    pltpu.sync_copy(ones_v.at[pl.ds(0, TAIL)], deg_sh.at[dst_t], add=True)
    plsc.subcore_barrier()

    def out_stripe(r0, nr):
        pltpu.sync_copy(deg_sh.at[pl.ds(r0, nr)], rows_v.at[pl.ds(0, nr)])
        pltpu.sync_copy(rows_v.at[pl.ds(0, nr)],
                        deg_out.at[pl.ds(cid * N + r0, nr)])

    _striped_rows(sid, out_stripe)


_SC_MESH = plsc.VectorSubcoreMesh(core_axis_name="c", subcore_axis_name="s")

_agg_call = pl.kernel(
    _sc_agg_body,
    out_type=jax.ShapeDtypeStruct((NC * N, D), jnp.float32),
    mesh=_SC_MESH,
    scratch_types=[
        pltpu.VMEM((K,), jnp.int32),
        pltpu.VMEM((K,), jnp.int32),
        pltpu.VMEM((K, D), jnp.float32),
        pltpu.VMEM((TAIL,), jnp.int32),
        pltpu.VMEM((TAIL,), jnp.int32),
        pltpu.VMEM((TAIL, D), jnp.float32),
        pltpu.SemaphoreType.DMA,
        pltpu.VMEM_SHARED((N, D), jnp.float32),
    ],
)

_deg_call = pl.kernel(
    _sc_deg_body,
    out_type=jax.ShapeDtypeStruct((NC * N, D), jnp.float32),
    mesh=_SC_MESH,
    scratch_types=[
        pltpu.VMEM((K,), jnp.int32),
        pltpu.VMEM((TAIL,), jnp.int32),
        pltpu.VMEM((K, D), jnp.float32),
        pltpu.VMEM((K, D), jnp.float32),
        pltpu.VMEM_SHARED((N, D), jnp.float32),
    ],
)

BN = 1000


def _tc_mlp_body(scale_ref, h_ref, a0_ref, a1_ref, d0_ref, d1_ref,
                 w1_ref, b1_ref, w2_ref, b2_ref, o_ref):
    deg = jnp.maximum(d0_ref[...] + d1_ref[...], 1.0)
    z = scale_ref[0, 0] * h_ref[...] + (a0_ref[...] + a1_ref[...]) / deg
    z = jnp.dot(z, w1_ref[...], preferred_element_type=jnp.float32) + b1_ref[...]
    z = jnp.where(z > 0, z, 0.01 * z)
    z = jnp.dot(z, w2_ref[...], preferred_element_type=jnp.float32) + b2_ref[...]
    o_ref[...] = jnp.where(z > 0, z, 0.01 * z)


_NB = N // BN

_tc_mlp_call = pl.pallas_call(
    _tc_mlp_body,
    grid=(_NB,),
    in_specs=[
        pl.BlockSpec(memory_space=pltpu.SMEM),
        pl.BlockSpec((BN, D), lambda i: (i, 0)),
        pl.BlockSpec((BN, D), lambda i: (i, 0)),
        pl.BlockSpec((BN, D), lambda i: (i + _NB, 0)),
        pl.BlockSpec((BN, D), lambda i: (i, 0)),
        pl.BlockSpec((BN, D), lambda i: (i + _NB, 0)),
        pl.BlockSpec((D, D), lambda i: (0, 0)),
        pl.BlockSpec((1, D), lambda i: (0, 0)),
        pl.BlockSpec((D, D), lambda i: (0, 0)),
        pl.BlockSpec((1, D), lambda i: (0, 0)),
    ],
    out_specs=pl.BlockSpec((BN, D), lambda i: (i, 0)),
    out_shape=jax.ShapeDtypeStruct((N, D), jnp.float32),
)


def kernel(x, edge_index,
           eps0, W1_0, b1_0, W2_0, b2_0,
           eps1, W1_1, b1_1, W2_1, b2_1,
           eps2, W1_2, b1_2, W2_2, b2_2):
    src = edge_index[0]
    dst = edge_index[1]
    zeros = jnp.zeros((N, D), jnp.float32)
    ones = jnp.ones((K, D), jnp.float32)

    def mlp(h, agg2, deg2, eps, W1, b1, W2, b2):
        scale = (1.0 + eps).reshape(1, 1)
        return _tc_mlp_call(scale, h, agg2, agg2, deg2, deg2,
                            W1, b1.reshape(1, D), W2, b2.reshape(1, D))

    deg2 = _deg_call(dst, zeros, ones)
    agg2 = _agg_call(x, src, dst, zeros)
    h = mlp(x, agg2, deg2, eps0, W1_0, b1_0, W2_0, b2_0)
    agg2 = _agg_call(h, src, dst, zeros)
    h = mlp(h, agg2, deg2, eps1, W1_1, b1_1, W2_1, b2_1)
    agg2 = _agg_call(h, src, dst, zeros)
    h = mlp(h, agg2, deg2, eps2, W1_2, b1_2, W2_2, b2_2)
    return h
